# baseline (speedup 1.0000x reference)
import functools

import jax
import jax.numpy as jnp
from jax import lax
from jax.experimental import pallas as pl
from jax.experimental.pallas import tpu as pltpu
from jax.experimental.pallas import tpu_sc as plsc

EMBED = 64
LANES = 16
NC, NS = 2, 16
NW = NC * NS
CHUNK = 128
SCALE = 8.0


@functools.partial(jax.jit, static_argnums=(2,))
def _embed_sc(tokens3, table, l_total):
    nw, nchunks, chunk = tokens3.shape
    bblocks = (nw * nchunks) // l_total
    mesh = plsc.VectorSubcoreMesh(core_axis_name="c", subcore_axis_name="s")

    @functools.partial(
        pl.kernel,
        mesh=mesh,
        compiler_params=pltpu.CompilerParams(
            use_tc_tiling_on_sc=False, needs_layout_passes=False),
        out_type=jax.ShapeDtypeStruct(
            (l_total, EMBED // 8, bblocks, 8, CHUNK), jnp.float32),
        scratch_types=[
            pltpu.VMEM((nchunks, chunk), jnp.int32),
            pltpu.VMEM((chunk, EMBED), jnp.float32),
            pltpu.VMEM((EMBED, chunk), jnp.float32),
            pltpu.SemaphoreType.DMA,
        ],
    )
    def k(tok_hbm, tab_hbm, out_hbm, idx_v, gbuf, obuf, sem):
        wid = lax.axis_index("s") * NC + lax.axis_index("c")
        pltpu.sync_copy(tok_hbm.at[wid], idx_v)
        rows = [lax.iota(jnp.int32, LANES) + bg * LANES
                for bg in range(chunk // LANES)]

        def chunk_body(j, carry):
            p = wid * nchunks + j
            l = p // bblocks
            bblk = p % bblocks
            pltpu.async_copy(tab_hbm.at[idx_v.at[j]], gbuf, sem).wait()

            def trans_body(e, c):
                ce = jnp.full((LANES,), e, jnp.int32)
                for bg in range(chunk // LANES):
                    v = plsc.load_gather(gbuf, [rows[bg], ce])
                    obuf[e, pl.ds(bg * LANES, LANES)] = v * SCALE
                return c

            lax.fori_loop(0, EMBED, trans_body, 0)
            for et in range(EMBED // 8):
                pltpu.sync_copy(obuf.at[pl.ds(et * 8, 8)],
                                out_hbm.at[l, et, bblk])
            return carry

        lax.fori_loop(0, nchunks, chunk_body, 0)

    return k(tokens3, table)


def kernel(tokens, input_embedding_table):
    b, l = tokens.shape
    n = b * l
    tokens3 = tokens.T.reshape(NW, n // (NW * CHUNK), CHUNK).astype(jnp.int32)
    out5 = _embed_sc(tokens3, input_embedding_table, l)
    return out5.transpose(2, 4, 0, 1, 3).reshape(b, l, EMBED)


# speedup vs baseline: 1.2045x; 1.2045x over previous
import functools

import jax
import jax.numpy as jnp
from jax import lax
from jax.experimental import pallas as pl
from jax.experimental.pallas import tpu as pltpu
from jax.experimental.pallas import tpu_sc as plsc

EMBED = 64
LANES = 16
NC, NS = 2, 16
NW = NC * NS
CHUNK = 128
SCALE = 8.0


@functools.partial(jax.jit, static_argnums=(2,))
def _embed_sc(tokens3, table, l_total):
    nw, nchunks, chunk = tokens3.shape
    bblocks = (nw * nchunks) // l_total
    mesh = plsc.VectorSubcoreMesh(core_axis_name="c", subcore_axis_name="s")

    @functools.partial(
        pl.kernel,
        mesh=mesh,
        compiler_params=pltpu.CompilerParams(
            use_tc_tiling_on_sc=False, needs_layout_passes=False),
        out_type=jax.ShapeDtypeStruct(
            (l_total, EMBED // 8, bblocks, 8, CHUNK), jnp.float32),
        scratch_types=[
            pltpu.VMEM((nchunks, chunk), jnp.int32),
            pltpu.VMEM((chunk, EMBED), jnp.float32),
            pltpu.VMEM((chunk, EMBED + 8), jnp.float32),
            pltpu.VMEM((EMBED, chunk), jnp.float32),
            pltpu.SemaphoreType.DMA,
        ],
    )
    def k(tok_hbm, tab_hbm, out_hbm, idx_v, gbuf, pbuf, obuf, sem):
        wid = lax.axis_index("s") * NC + lax.axis_index("c")
        pltpu.sync_copy(tok_hbm.at[wid], idx_v)
        rows = [lax.iota(jnp.int32, LANES) + bg * LANES
                for bg in range(chunk // LANES)]

        def chunk_body(j, carry):
            p = wid * nchunks + j
            l = p // bblocks
            bblk = p % bblocks
            pltpu.async_copy(tab_hbm.at[idx_v.at[j]], gbuf, sem).wait()

            def relay_body(r, c):
                for kk in range(EMBED // LANES):
                    sl = pl.ds(kk * LANES, LANES)
                    pbuf[r, sl] = gbuf[r, sl] * SCALE
                return c

            lax.fori_loop(0, chunk, relay_body, 0, unroll=2)

            def trans_body(e, c):
                ce = jnp.full((LANES,), e, jnp.int32)
                for bg in range(chunk // LANES):
                    v = plsc.load_gather(pbuf, [rows[bg], ce])
                    obuf[e, pl.ds(bg * LANES, LANES)] = v
                return c

            lax.fori_loop(0, EMBED, trans_body, 0)
            for et in range(EMBED // 8):
                pltpu.sync_copy(obuf.at[pl.ds(et * 8, 8)],
                                out_hbm.at[l, et, bblk])
            return carry

        lax.fori_loop(0, nchunks, chunk_body, 0)

    return k(tokens3, table)


def kernel(tokens, input_embedding_table):
    b, l = tokens.shape
    n = b * l
    tokens3 = tokens.T.reshape(NW, n // (NW * CHUNK), CHUNK).astype(jnp.int32)
    out5 = _embed_sc(tokens3, input_embedding_table, l)
    return out5.transpose(2, 4, 0, 1, 3).reshape(b, l, EMBED)


# speedup vs baseline: 2.4048x; 1.9965x over previous
import functools

import jax
import jax.numpy as jnp
from jax import lax
from jax.experimental import pallas as pl
from jax.experimental.pallas import tpu as pltpu
from jax.experimental.pallas import tpu_sc as plsc

EMBED = 64
LANES = 16
NC, NS = 2, 16
NW = NC * NS
CHUNK = 128
OPITCH = CHUNK + 8
SCALE = 8.0


@functools.partial(jax.jit, static_argnums=(2,))
def _embed_sc(tokens3, table, l_total):
    nw, nchunks, chunk = tokens3.shape
    bblocks = (nw * nchunks) // l_total
    mesh = plsc.VectorSubcoreMesh(core_axis_name="c", subcore_axis_name="s")

    scratch = [pltpu.VMEM((nchunks, chunk), jnp.int32)]
    scratch += [pltpu.VMEM((chunk, EMBED), jnp.float32) for _ in range(2)]
    scratch += [pltpu.VMEM((EMBED, OPITCH), jnp.float32) for _ in range(2)]
    scratch += [pltpu.SemaphoreType.DMA for _ in range(4)]

    @functools.partial(
        pl.kernel,
        mesh=mesh,
        compiler_params=pltpu.CompilerParams(
            use_tc_tiling_on_sc=False, needs_layout_passes=False),
        out_type=jax.ShapeDtypeStruct(
            (l_total, EMBED // 8, bblocks, 8, CHUNK), jnp.float32),
        scratch_types=scratch,
    )
    def k(tok_hbm, tab_hbm, out_hbm, idx_v, g0, g1, o0, o1, gs0, gs1,
          os0, os1):
        gbuf, obuf = (g0, g1), (o0, o1)
        gsem, osem = (gs0, gs1), (os0, os1)
        wid = lax.axis_index("s") * NC + lax.axis_index("c")
        pltpu.sync_copy(tok_hbm.at[wid], idx_v)
        scat_rows = [lax.iota(jnp.int32, LANES) + kk * LANES
                     for kk in range(EMBED // LANES)]
        pltpu.async_copy(tab_hbm.at[idx_v.at[0]], gbuf[0], gsem[0])

        def outer(g, carry):
            for b in range(2):
                j = g * 2 + b
                p = wid * nchunks + j
                l = p // bblocks
                bblk = p % bblocks
                pltpu.make_async_copy(
                    tab_hbm.at[idx_v.at[j]], gbuf[b], gsem[b]).wait()

                @pl.when(j + 1 < nchunks)
                def _fire_next(b=b, j=j):
                    pltpu.async_copy(
                        tab_hbm.at[idx_v.at[j + 1]], gbuf[1 - b],
                        gsem[1 - b])

                @pl.when(g > 0)
                def _drain_out(b=b):
                    for et in range(EMBED // 8):
                        pltpu.make_async_copy(
                            obuf[b].at[pl.ds(et * 8, 8), pl.ds(0, CHUNK)],
                            out_hbm.at[0, et, 0], osem[b]).wait()

                def relay_body(r, c, b=b):
                    cr = jnp.full((LANES,), r, jnp.int32)
                    for kk in range(EMBED // LANES):
                        v = gbuf[b][r, pl.ds(kk * LANES, LANES)] * SCALE
                        plsc.store_scatter(obuf[b], [scat_rows[kk], cr], v)
                    return c

                lax.fori_loop(0, chunk, relay_body, 0, unroll=2)
                for et in range(EMBED // 8):
                    pltpu.async_copy(
                        obuf[b].at[pl.ds(et * 8, 8), pl.ds(0, CHUNK)],
                        out_hbm.at[l, et, bblk], osem[b])
            return carry

        lax.fori_loop(0, nchunks // 2, outer, 0)
        for b in range(2):
            for et in range(EMBED // 8):
                pltpu.make_async_copy(
                    obuf[b].at[pl.ds(et * 8, 8), pl.ds(0, CHUNK)],
                    out_hbm.at[0, et, 0], osem[b]).wait()

    return k(tokens3, table)


def kernel(tokens, input_embedding_table):
    b, l = tokens.shape
    n = b * l
    tokens3 = tokens.T.reshape(NW, n // (NW * CHUNK), CHUNK).astype(jnp.int32)
    out5 = _embed_sc(tokens3, input_embedding_table, l)
    return out5.transpose(2, 4, 0, 1, 3).reshape(b, l, EMBED)


# speedup vs baseline: 2.4303x; 1.0106x over previous
import functools

import jax
import jax.numpy as jnp
from jax import lax
from jax.experimental import pallas as pl
from jax.experimental.pallas import tpu as pltpu
from jax.experimental.pallas import tpu_sc as plsc

EMBED = 64
LANES = 16
NC, NS = 2, 16
NW = NC * NS
CHUNK = 128
OPITCH = CHUNK + 8
SCALE = 8.0


@functools.partial(jax.jit, static_argnums=(2,))
def _embed_sc(tokens3, table, l_total):
    nw, nchunks, chunk = tokens3.shape
    bblocks = (nw * nchunks) // l_total
    mesh = plsc.VectorSubcoreMesh(core_axis_name="c", subcore_axis_name="s")

    scratch = [pltpu.VMEM((nchunks, chunk), jnp.int32)]
    scratch += [pltpu.VMEM((chunk, EMBED), jnp.float32) for _ in range(2)]
    scratch += [pltpu.VMEM((EMBED // 8, 8, OPITCH), jnp.float32)
                for _ in range(2)]
    scratch += [pltpu.SemaphoreType.DMA for _ in range(4)]

    @functools.partial(
        pl.kernel,
        mesh=mesh,
        compiler_params=pltpu.CompilerParams(
            use_tc_tiling_on_sc=False, needs_layout_passes=False),
        out_type=jax.ShapeDtypeStruct(
            (l_total, EMBED // 8, bblocks, 8, CHUNK), jnp.float32),
        scratch_types=scratch,
    )
    def k(tok_hbm, tab_hbm, out_hbm, idx_v, g0, g1, o0, o1, gs0, gs1,
          os0, os1):
        gbuf, obuf = (g0, g1), (o0, o1)
        gsem, osem = (gs0, gs1), (os0, os1)
        wid = lax.axis_index("s") * NC + lax.axis_index("c")
        pltpu.sync_copy(tok_hbm.at[wid], idx_v)
        lanes = lax.iota(jnp.int32, LANES)
        scat_et = [lanes // 8 + kk * 2 for kk in range(EMBED // LANES)]
        scat_ei = lanes % 8
        pltpu.async_copy(tab_hbm.at[idx_v.at[0]], gbuf[0], gsem[0])

        def outer(g, carry):
            for b in range(2):
                j = g * 2 + b
                q = wid * nchunks + j
                lt = q // (bblocks * 8)
                rem = q % (bblocks * 8)
                bblk = rem // 8
                l = lt * 8 + rem % 8
                pltpu.make_async_copy(
                    tab_hbm.at[idx_v.at[j]], gbuf[b], gsem[b]).wait()

                @pl.when(j + 1 < nchunks)
                def _fire_next(b=b, j=j):
                    pltpu.async_copy(
                        tab_hbm.at[idx_v.at[j + 1]], gbuf[1 - b],
                        gsem[1 - b])

                @pl.when(g > 0)
                def _drain_out(b=b):
                    pltpu.make_async_copy(
                        obuf[b].at[:, :, pl.ds(0, CHUNK)],
                        out_hbm.at[0, :, 0], osem[b]).wait()

                def relay_body(r, c, b=b):
                    cr = jnp.full((LANES,), r, jnp.int32)
                    for kk in range(EMBED // LANES):
                        v = gbuf[b][r, pl.ds(kk * LANES, LANES)] * SCALE
                        plsc.store_scatter(
                            obuf[b], [scat_et[kk], scat_ei, cr], v)
                    return c

                lax.fori_loop(0, chunk, relay_body, 0, unroll=2)
                pltpu.async_copy(
                    obuf[b].at[:, :, pl.ds(0, CHUNK)],
                    out_hbm.at[l, :, bblk], osem[b])
            return carry

        lax.fori_loop(0, nchunks // 2, outer, 0)
        for b in range(2):
            pltpu.make_async_copy(
                obuf[b].at[:, :, pl.ds(0, CHUNK)],
                out_hbm.at[0, :, 0], osem[b]).wait()

    return k(tokens3, table)


def kernel(tokens, input_embedding_table):
    b, l = tokens.shape
    n = b * l
    tokens3 = (tokens.T.reshape(l // 8, 8, b // CHUNK, CHUNK)
               .transpose(0, 2, 1, 3)
               .reshape(NW, n // (NW * CHUNK), CHUNK).astype(jnp.int32))
    out5 = _embed_sc(tokens3, input_embedding_table, l)
    return out5.transpose(2, 4, 0, 1, 3).reshape(b, l, EMBED)


# speedup vs baseline: 2.4427x; 1.0051x over previous
import functools

import jax
import jax.numpy as jnp
from jax import lax
from jax.experimental import pallas as pl
from jax.experimental.pallas import tpu as pltpu
from jax.experimental.pallas import tpu_sc as plsc

EMBED = 64
LANES = 16
NC, NS = 2, 16
NW = NC * NS
CHUNK = 128
OPITCH = CHUNK + 8
SCALE = 8.0


@functools.partial(jax.jit, static_argnums=(2,))
def _embed_sc(tokens3, table, l_total):
    nw, nchunks, chunk = tokens3.shape
    bblocks = (nw * nchunks) // l_total
    mesh = plsc.VectorSubcoreMesh(core_axis_name="c", subcore_axis_name="s")

    scratch = [pltpu.VMEM((nchunks, chunk), jnp.int32)]
    scratch += [pltpu.VMEM((chunk, EMBED), jnp.float32) for _ in range(2)]
    scratch += [pltpu.VMEM((EMBED // 8, 8, OPITCH), jnp.float32)
                for _ in range(2)]
    scratch += [pltpu.SemaphoreType.DMA for _ in range(4)]

    @functools.partial(
        pl.kernel,
        mesh=mesh,
        compiler_params=pltpu.CompilerParams(
            use_tc_tiling_on_sc=False, needs_layout_passes=False),
        out_type=jax.ShapeDtypeStruct(
            (l_total, EMBED // 8, bblocks, 8, CHUNK), jnp.float32),
        scratch_types=scratch,
    )
    def k(tok_hbm, tab_hbm, out_hbm, idx_v, g0, g1, o0, o1, gs0, gs1,
          os0, os1):
        gbuf, obuf = (g0, g1), (o0, o1)
        gsem, osem = (gs0, gs1), (os0, os1)
        wid = lax.axis_index("s") * NC + lax.axis_index("c")
        pltpu.sync_copy(tok_hbm.at[wid], idx_v)
        lanes = lax.iota(jnp.int32, LANES)
        scat_et = [lanes // 8 + kk * 2 for kk in range(EMBED // LANES)]
        scat_ei = lanes % 8
        pltpu.async_copy(tab_hbm.at[idx_v.at[0]], gbuf[0], gsem[0])

        def outer(g, carry):
            for b in range(2):
                j = g * 2 + b
                q = wid * nchunks + j
                lt = q // (bblocks * 8)
                rem = q % (bblocks * 8)
                bblk = rem // 8
                l = lt * 8 + rem % 8
                pltpu.make_async_copy(
                    tab_hbm.at[idx_v.at[j]], gbuf[b], gsem[b]).wait()

                @pl.when(j + 1 < nchunks)
                def _fire_next(b=b, j=j):
                    pltpu.async_copy(
                        tab_hbm.at[idx_v.at[j + 1]], gbuf[1 - b],
                        gsem[1 - b])

                @pl.when(g > 0)
                def _drain_out(b=b):
                    pltpu.make_async_copy(
                        obuf[b].at[:, :, pl.ds(0, CHUNK)],
                        out_hbm.at[0, :, 0], osem[b]).wait()

                def relay_body(r, c, b=b):
                    cr = jnp.full((LANES,), r, jnp.int32)
                    for kk in range(EMBED // LANES):
                        v = gbuf[b][r, pl.ds(kk * LANES, LANES)] * SCALE
                        plsc.store_scatter(
                            obuf[b], [scat_et[kk], scat_ei, cr], v)
                    return c

                lax.fori_loop(0, chunk, relay_body, 0, unroll=8)
                pltpu.async_copy(
                    obuf[b].at[:, :, pl.ds(0, CHUNK)],
                    out_hbm.at[l, :, bblk], osem[b])
            return carry

        lax.fori_loop(0, nchunks // 2, outer, 0)
        for b in range(2):
            pltpu.make_async_copy(
                obuf[b].at[:, :, pl.ds(0, CHUNK)],
                out_hbm.at[0, :, 0], osem[b]).wait()

    return k(tokens3, table)


def kernel(tokens, input_embedding_table):
    b, l = tokens.shape
    n = b * l
    tokens3 = (tokens.T.reshape(l // 8, 8, b // CHUNK, CHUNK)
               .transpose(0, 2, 1, 3)
               .reshape(NW, n // (NW * CHUNK), CHUNK).astype(jnp.int32))
    out5 = _embed_sc(tokens3, input_embedding_table, l)
    return out5.transpose(2, 4, 0, 1, 3).reshape(b, l, EMBED)
